# baseline (speedup 1.0000x reference)
import functools

import jax
import jax.numpy as jnp
from jax import lax
from jax.experimental import pallas as pl
from jax.experimental.pallas import tpu as pltpu
from jax.experimental.pallas import tpu_sc as plsc

N = 10000
NP = 10112
DEG_W = 8
E = 320000
FEAT = 128
EMB = 64
HID = 128
OUT = 64

NC = 2
NS = 16
NW = NC * NS
EDGES_PER_W = E // NW
CHUNK = 125
CHUNKS_PER_W = EDGES_PER_W // CHUNK
IDXB = 16
ROWS_PER_SUB = NP // NS

_MESH = plsc.VectorSubcoreMesh(
    core_axis_name="c", subcore_axis_name="s", num_cores=NC, num_subcores=NS
)


def _sc_degree(dst2d, zeros_deg, ones_tile):
    @functools.partial(
        pl.kernel,
        out_type=jax.ShapeDtypeStruct((NC, NP, DEG_W), jnp.float32),
        mesh=_MESH,
        scratch_types=[
            pltpu.VMEM((IDXB, CHUNK), jnp.int32),
            pltpu.VMEM((CHUNK, DEG_W), jnp.float32),
            pltpu.VMEM_SHARED((NP, DEG_W), jnp.float32),
            pltpu.SemaphoreType.DMA,
            pltpu.SemaphoreType.DMA,
        ],
    )
    def deg_kernel(dst_hbm, zdeg_hbm, ones_hbm, deg_hbm,
                   dst_v, ones_v, sh_deg, sem_a, sem_b):
        cid = lax.axis_index("c")
        sid = lax.axis_index("s")
        g = cid * NS + sid

        pltpu.sync_copy(
            zdeg_hbm, sh_deg.at[pl.ds(sid * ROWS_PER_SUB, ROWS_PER_SUB)]
        )
        pltpu.sync_copy(ones_hbm, ones_v)
        plsc.subcore_barrier()

        @pl.loop(0, CHUNKS_PER_W // IDXB)
        def _(blk):
            pltpu.sync_copy(dst_hbm.at[g, pl.ds(blk * IDXB, IDXB)], dst_v)

            @pl.loop(0, IDXB, step=2)
            def _(c):
                d1 = pltpu.async_copy(
                    ones_v, sh_deg.at[dst_v.at[c]], sem_a, add=True
                )
                d2 = pltpu.async_copy(
                    ones_v, sh_deg.at[dst_v.at[c + 1]], sem_b, add=True
                )
                d1.wait()
                d2.wait()

        plsc.subcore_barrier()
        sl = pl.ds(sid * ROWS_PER_SUB, ROWS_PER_SUB)
        pltpu.sync_copy(sh_deg.at[sl], deg_hbm.at[cid, sl])

    return deg_kernel(dst2d, zeros_deg, ones_tile)


def _sc_segment_sum(p, src1d, dst1d, zeros_d, nbuf, chunk, idxb, tc_tiling):
    D = p.shape[1]
    chunks_per_w = EDGES_PER_W // chunk
    src2d = src1d.reshape(NW, chunks_per_w, chunk)
    dst2d = dst1d.reshape(NW, chunks_per_w, chunk)

    scratch = [
        pltpu.VMEM((idxb, chunk), jnp.int32),
        pltpu.VMEM((idxb, chunk), jnp.int32),
    ]
    scratch += [pltpu.VMEM((chunk, D), jnp.float32) for _ in range(nbuf)]
    scratch += [pltpu.VMEM_SHARED((NP, D), jnp.float32)]
    scratch += [pltpu.SemaphoreType.DMA for _ in range(nbuf)]

    @functools.partial(
        pl.kernel,
        out_type=jax.ShapeDtypeStruct((NC, NP, D), jnp.float32),
        mesh=_MESH,
        scratch_types=scratch,
        compiler_params=None if tc_tiling else pltpu.CompilerParams(
            use_tc_tiling_on_sc=False),
    )
    def seg_kernel(p_hbm, src_hbm, dst_hbm, z_hbm, agg_hbm,
                   src_v, dst_v, *rows_and_sems):
        rows = rows_and_sems[:nbuf]
        sh_agg = rows_and_sems[nbuf]
        sems = rows_and_sems[nbuf + 1:]
        cid = lax.axis_index("c")
        sid = lax.axis_index("s")
        g = cid * NS + sid

        pltpu.sync_copy(z_hbm, sh_agg.at[pl.ds(sid * ROWS_PER_SUB, ROWS_PER_SUB)])
        plsc.subcore_barrier()

        def prime(blk):
            pltpu.sync_copy(src_hbm.at[g, pl.ds(blk * idxb, idxb)], src_v)
            pltpu.sync_copy(dst_hbm.at[g, pl.ds(blk * idxb, idxb)], dst_v)
            for b in range(nbuf):
                pltpu.async_copy(p_hbm.at[src_v.at[b]], rows[b], sems[b])

        prime(0)

        @pl.loop(0, chunks_per_w // idxb)
        def _(blk):
            @pl.loop(0, idxb, step=nbuf)
            def _(c):
                for b in range(nbuf):
                    j = c + b
                    pltpu.make_async_copy(
                        p_hbm.at[src_v.at[j]], rows[b], sems[b]
                    ).wait()
                    pltpu.sync_copy(rows[b], sh_agg.at[dst_v.at[j]], add=True)

                    @pl.when(j < idxb - nbuf)
                    def _():
                        pltpu.async_copy(
                            p_hbm.at[src_v.at[j + nbuf]], rows[b], sems[b]
                        )

            @pl.when(blk < chunks_per_w // idxb - 1)
            def _():
                prime(blk + 1)

        plsc.subcore_barrier()

        sl = pl.ds(sid * ROWS_PER_SUB, ROWS_PER_SUB)
        pltpu.sync_copy(sh_agg.at[sl], agg_hbm.at[cid, sl])

    return seg_kernel(p, src2d, dst2d, zeros_d)


ROW_BLK = 632
GRID = NP // ROW_BLK


def _matT(a, w):
    return lax.dot_general(a, w, (((1,), (1,)), ((), ())),
                           preferred_element_type=jnp.float32)


def _stage1(x, emb, W1_l, W1_r, b1):
    def body(x_ref, e_ref, wl_ref, wr_ref, b_ref, p1_ref, r1_ref):
        xv = x_ref[...]
        ev = e_ref[...]
        wl = wl_ref[...]
        wr = wr_ref[...]
        p1_ref[...] = _matT(xv, wl[:, :FEAT]) + _matT(ev, wl[:, FEAT:])
        r1_ref[...] = (_matT(xv, wr[:, :FEAT]) + _matT(ev, wr[:, FEAT:])
                       + b_ref[...])

    full = lambda s: pl.BlockSpec(s, lambda i: (0,) * len(s))
    return pl.pallas_call(
        body,
        grid=(GRID,),
        in_specs=[
            pl.BlockSpec((ROW_BLK, FEAT), lambda i: (i, 0)),
            pl.BlockSpec((ROW_BLK, EMB), lambda i: (i, 0)),
            full((HID, FEAT + EMB)), full((HID, FEAT + EMB)),
            full((1, HID)),
        ],
        out_specs=[
            pl.BlockSpec((ROW_BLK, HID), lambda i: (i, 0)),
            pl.BlockSpec((ROW_BLK, HID), lambda i: (i, 0)),
        ],
        out_shape=[
            jax.ShapeDtypeStruct((NP, HID), jnp.float32),
            jax.ShapeDtypeStruct((NP, HID), jnp.float32),
        ],
    )(x, emb, W1_l, W1_r, b1)


def _stage2(aggp, degp, r1, w2l, w2r, b2):
    def body(agg_ref, deg_ref, r1_ref, w2l_ref, w2r_ref, b_ref,
             p2_ref, r2_ref):
        aggs = agg_ref[...]
        degs = deg_ref[...]
        deg = degs[0, :, 0:1] + degs[1, :, 0:1]
        inv = 1.0 / jnp.maximum(deg, 1.0)
        h1 = jnp.maximum((aggs[0] + aggs[1]) * inv + r1_ref[...], 0.0)
        p2_ref[...] = _matT(h1, w2l_ref[...])
        r2_ref[...] = _matT(h1, w2r_ref[...]) + b_ref[...]

    full = lambda s: pl.BlockSpec(s, lambda i: (0,) * len(s))
    return pl.pallas_call(
        body,
        grid=(GRID,),
        in_specs=[
            pl.BlockSpec((NC, ROW_BLK, HID), lambda i: (0, i, 0)),
            pl.BlockSpec((NC, ROW_BLK, DEG_W), lambda i: (0, i, 0)),
            pl.BlockSpec((ROW_BLK, HID), lambda i: (i, 0)),
            full((OUT, HID)), full((OUT, HID)), full((1, OUT)),
        ],
        out_specs=[
            pl.BlockSpec((ROW_BLK, OUT), lambda i: (i, 0)),
            pl.BlockSpec((ROW_BLK, OUT), lambda i: (i, 0)),
        ],
        out_shape=[
            jax.ShapeDtypeStruct((NP, OUT), jnp.float32),
            jax.ShapeDtypeStruct((NP, OUT), jnp.float32),
        ],
    )(aggp, degp, r1, w2l, w2r, b2)


def _stage3(agg2p, degp, r2):
    def body(agg_ref, deg_ref, r2_ref, out_ref):
        aggs = agg_ref[...]
        degs = deg_ref[...]
        deg = degs[0, :, 0:1] + degs[1, :, 0:1]
        inv = 1.0 / jnp.maximum(deg, 1.0)
        out_ref[...] = (aggs[0] + aggs[1]) * inv + r2_ref[...]

    return pl.pallas_call(
        body,
        grid=(GRID,),
        in_specs=[
            pl.BlockSpec((NC, ROW_BLK, OUT), lambda i: (0, i, 0)),
            pl.BlockSpec((NC, ROW_BLK, DEG_W), lambda i: (0, i, 0)),
            pl.BlockSpec((ROW_BLK, OUT), lambda i: (i, 0)),
        ],
        out_specs=pl.BlockSpec((ROW_BLK, OUT), lambda i: (i, 0)),
        out_shape=jax.ShapeDtypeStruct((N, OUT), jnp.float32),
    )(agg2p, degp, r2)


@jax.jit
def kernel(x, edge_index, emb, W1_l, b1_l, W1_r, W2_l, b2_l, W2_r):
    src1d = edge_index[0].astype(jnp.int32)
    dst1d = edge_index[1].astype(jnp.int32)

    zeros_deg = jnp.zeros((ROWS_PER_SUB, DEG_W), jnp.float32)
    ones_tile = jnp.ones((CHUNK, DEG_W), jnp.float32)
    degp = _sc_degree(dst1d.reshape(NW, CHUNKS_PER_W, CHUNK), zeros_deg, ones_tile)

    p1, r1 = _stage1(x, emb, W1_l, W1_r, b1_l.reshape(1, HID))

    zeros_hid = jnp.zeros((ROWS_PER_SUB, HID), jnp.float32)
    aggp = _sc_segment_sum(p1, src1d, dst1d, zeros_hid, nbuf=2,
                           chunk=CHUNK, idxb=IDXB, tc_tiling=True)

    p2, r2 = _stage2(aggp, degp, r1, W2_l, W2_r, b2_l.reshape(1, OUT))

    zeros_out = jnp.zeros((ROWS_PER_SUB, OUT), jnp.float32)
    agg2p = _sc_segment_sum(p2, src1d, dst1d, zeros_out, nbuf=4,
                            chunk=CHUNK, idxb=IDXB, tc_tiling=False)

    return _stage3(agg2p, degp, r2)


# speedup vs baseline: 1.0579x; 1.0579x over previous
import functools

import jax
import jax.numpy as jnp
from jax import lax
from jax.experimental import pallas as pl
from jax.experimental.pallas import tpu as pltpu
from jax.experimental.pallas import tpu_sc as plsc

N = 10000
NP = 10112
DEG_W = 8
E = 320000
FEAT = 128
EMB = 64
HID = 128
OUT = 64

NC = 2
NS = 16
NW = NC * NS
EDGES_PER_W = E // NW
CHUNK = 125
CHUNKS_PER_W = EDGES_PER_W // CHUNK
IDXB = 16
ROWS_PER_SUB = NP // NS

_MESH = plsc.VectorSubcoreMesh(
    core_axis_name="c", subcore_axis_name="s", num_cores=NC, num_subcores=NS
)


def _sc_degree(dst2d, zeros_deg, ones_tile):
    @functools.partial(
        pl.kernel,
        out_type=jax.ShapeDtypeStruct((NC, NP, DEG_W), jnp.float32),
        mesh=_MESH,
        scratch_types=[
            pltpu.VMEM((IDXB, CHUNK), jnp.int32),
            pltpu.VMEM((CHUNK, DEG_W), jnp.float32),
            pltpu.VMEM_SHARED((NP, DEG_W), jnp.float32),
            pltpu.SemaphoreType.DMA,
            pltpu.SemaphoreType.DMA,
        ],
        compiler_params=pltpu.CompilerParams(use_tc_tiling_on_sc=False),
    )
    def deg_kernel(dst_hbm, zdeg_hbm, ones_hbm, deg_hbm,
                   dst_v, ones_v, sh_deg, sem_a, sem_b):
        cid = lax.axis_index("c")
        sid = lax.axis_index("s")
        g = cid * NS + sid

        pltpu.sync_copy(
            zdeg_hbm, sh_deg.at[pl.ds(sid * ROWS_PER_SUB, ROWS_PER_SUB)]
        )
        pltpu.sync_copy(ones_hbm, ones_v)
        plsc.subcore_barrier()

        @pl.loop(0, CHUNKS_PER_W // IDXB)
        def _(blk):
            pltpu.sync_copy(dst_hbm.at[g, pl.ds(blk * IDXB, IDXB)], dst_v)

            @pl.loop(0, IDXB, step=2)
            def _(c):
                d1 = pltpu.async_copy(
                    ones_v, sh_deg.at[dst_v.at[c]], sem_a, add=True
                )
                d2 = pltpu.async_copy(
                    ones_v, sh_deg.at[dst_v.at[c + 1]], sem_b, add=True
                )
                d1.wait()
                d2.wait()

        plsc.subcore_barrier()
        sl = pl.ds(sid * ROWS_PER_SUB, ROWS_PER_SUB)
        pltpu.sync_copy(sh_deg.at[sl], deg_hbm.at[cid, sl])

    return deg_kernel(dst2d, zeros_deg, ones_tile)


def _sc_segment_sum(p, src1d, dst1d, zeros_d, nbuf, chunk, idxb, tc_tiling):
    D = p.shape[1]
    chunks_per_w = EDGES_PER_W // chunk
    src2d = src1d.reshape(NW, chunks_per_w, chunk)
    dst2d = dst1d.reshape(NW, chunks_per_w, chunk)

    scratch = [
        pltpu.VMEM((idxb, chunk), jnp.int32),
        pltpu.VMEM((idxb, chunk), jnp.int32),
    ]
    scratch += [pltpu.VMEM((chunk, D), jnp.float32) for _ in range(nbuf)]
    scratch += [pltpu.VMEM_SHARED((NP, D), jnp.float32)]
    scratch += [pltpu.SemaphoreType.DMA for _ in range(nbuf)]

    @functools.partial(
        pl.kernel,
        out_type=jax.ShapeDtypeStruct((NC, NP, D), jnp.float32),
        mesh=_MESH,
        scratch_types=scratch,
        compiler_params=None if tc_tiling else pltpu.CompilerParams(
            use_tc_tiling_on_sc=False),
    )
    def seg_kernel(p_hbm, src_hbm, dst_hbm, z_hbm, agg_hbm,
                   src_v, dst_v, *rows_and_sems):
        rows = rows_and_sems[:nbuf]
        sh_agg = rows_and_sems[nbuf]
        sems = rows_and_sems[nbuf + 1:]
        cid = lax.axis_index("c")
        sid = lax.axis_index("s")
        g = cid * NS + sid

        pltpu.sync_copy(z_hbm, sh_agg.at[pl.ds(sid * ROWS_PER_SUB, ROWS_PER_SUB)])
        plsc.subcore_barrier()

        def prime(blk):
            pltpu.sync_copy(src_hbm.at[g, pl.ds(blk * idxb, idxb)], src_v)
            pltpu.sync_copy(dst_hbm.at[g, pl.ds(blk * idxb, idxb)], dst_v)
            for b in range(nbuf):
                pltpu.async_copy(p_hbm.at[src_v.at[b]], rows[b], sems[b])

        prime(0)

        @pl.loop(0, chunks_per_w // idxb)
        def _(blk):
            @pl.loop(0, idxb, step=nbuf)
            def _(c):
                for b in range(nbuf):
                    j = c + b
                    pltpu.make_async_copy(
                        p_hbm.at[src_v.at[j]], rows[b], sems[b]
                    ).wait()
                    pltpu.sync_copy(rows[b], sh_agg.at[dst_v.at[j]], add=True)

                    @pl.when(j < idxb - nbuf)
                    def _():
                        pltpu.async_copy(
                            p_hbm.at[src_v.at[j + nbuf]], rows[b], sems[b]
                        )

            @pl.when(blk < chunks_per_w // idxb - 1)
            def _():
                prime(blk + 1)

        plsc.subcore_barrier()

        sl = pl.ds(sid * ROWS_PER_SUB, ROWS_PER_SUB)
        pltpu.sync_copy(sh_agg.at[sl], agg_hbm.at[cid, sl])

    return seg_kernel(p, src2d, dst2d, zeros_d)


ROW_BLK = 632
GRID = NP // ROW_BLK


def _matT(a, w):
    return lax.dot_general(a, w, (((1,), (1,)), ((), ())),
                           preferred_element_type=jnp.float32)


def _stage1(x, emb, W1_l, W1_r, b1):
    def body(x_ref, e_ref, wl_ref, wr_ref, b_ref, p1_ref, r1_ref):
        xv = x_ref[...]
        ev = e_ref[...]
        wl = wl_ref[...]
        wr = wr_ref[...]
        p1_ref[...] = _matT(xv, wl[:, :FEAT]) + _matT(ev, wl[:, FEAT:])
        r1_ref[...] = (_matT(xv, wr[:, :FEAT]) + _matT(ev, wr[:, FEAT:])
                       + b_ref[...])

    full = lambda s: pl.BlockSpec(s, lambda i: (0,) * len(s))
    return pl.pallas_call(
        body,
        grid=(GRID,),
        in_specs=[
            pl.BlockSpec((ROW_BLK, FEAT), lambda i: (i, 0)),
            pl.BlockSpec((ROW_BLK, EMB), lambda i: (i, 0)),
            full((HID, FEAT + EMB)), full((HID, FEAT + EMB)),
            full((1, HID)),
        ],
        out_specs=[
            pl.BlockSpec((ROW_BLK, HID), lambda i: (i, 0)),
            pl.BlockSpec((ROW_BLK, HID), lambda i: (i, 0)),
        ],
        out_shape=[
            jax.ShapeDtypeStruct((NP, HID), jnp.float32),
            jax.ShapeDtypeStruct((NP, HID), jnp.float32),
        ],
    )(x, emb, W1_l, W1_r, b1)


def _stage2(aggp, degp, r1, w2l, w2r, b2):
    def body(agg_ref, deg_ref, r1_ref, w2l_ref, w2r_ref, b_ref,
             p2_ref, r2_ref):
        aggs = agg_ref[...]
        degs = deg_ref[...]
        deg = degs[0, :, 0:1] + degs[1, :, 0:1]
        inv = 1.0 / jnp.maximum(deg, 1.0)
        h1 = jnp.maximum((aggs[0] + aggs[1]) * inv + r1_ref[...], 0.0)
        p2_ref[...] = _matT(h1, w2l_ref[...])
        r2_ref[...] = _matT(h1, w2r_ref[...]) + b_ref[...]

    full = lambda s: pl.BlockSpec(s, lambda i: (0,) * len(s))
    return pl.pallas_call(
        body,
        grid=(GRID,),
        in_specs=[
            pl.BlockSpec((NC, ROW_BLK, HID), lambda i: (0, i, 0)),
            pl.BlockSpec((NC, ROW_BLK, DEG_W), lambda i: (0, i, 0)),
            pl.BlockSpec((ROW_BLK, HID), lambda i: (i, 0)),
            full((OUT, HID)), full((OUT, HID)), full((1, OUT)),
        ],
        out_specs=[
            pl.BlockSpec((ROW_BLK, OUT), lambda i: (i, 0)),
            pl.BlockSpec((ROW_BLK, OUT), lambda i: (i, 0)),
        ],
        out_shape=[
            jax.ShapeDtypeStruct((NP, OUT), jnp.float32),
            jax.ShapeDtypeStruct((NP, OUT), jnp.float32),
        ],
    )(aggp, degp, r1, w2l, w2r, b2)


def _stage3(agg2p, degp, r2):
    def body(agg_ref, deg_ref, r2_ref, out_ref):
        aggs = agg_ref[...]
        degs = deg_ref[...]
        deg = degs[0, :, 0:1] + degs[1, :, 0:1]
        inv = 1.0 / jnp.maximum(deg, 1.0)
        out_ref[...] = (aggs[0] + aggs[1]) * inv + r2_ref[...]

    return pl.pallas_call(
        body,
        grid=(GRID,),
        in_specs=[
            pl.BlockSpec((NC, ROW_BLK, OUT), lambda i: (0, i, 0)),
            pl.BlockSpec((NC, ROW_BLK, DEG_W), lambda i: (0, i, 0)),
            pl.BlockSpec((ROW_BLK, OUT), lambda i: (i, 0)),
        ],
        out_specs=pl.BlockSpec((ROW_BLK, OUT), lambda i: (i, 0)),
        out_shape=jax.ShapeDtypeStruct((NP, OUT), jnp.float32),
    )(agg2p, degp, r2)


@jax.jit
def kernel(x, edge_index, emb, W1_l, b1_l, W1_r, W2_l, b2_l, W2_r):
    src1d = edge_index[0].astype(jnp.int32)
    dst1d = edge_index[1].astype(jnp.int32)

    zeros_deg = jnp.zeros((ROWS_PER_SUB, DEG_W), jnp.float32)
    ones_tile = jnp.ones((CHUNK, DEG_W), jnp.float32)
    degp = _sc_degree(dst1d.reshape(NW, CHUNKS_PER_W, CHUNK), zeros_deg, ones_tile)

    xp = jnp.pad(x, ((0, NP - N), (0, 0)))
    ep = jnp.pad(emb, ((0, NP - N), (0, 0)))
    p1, r1 = _stage1(xp, ep, W1_l, W1_r, b1_l.reshape(1, HID))

    zeros_hid = jnp.zeros((ROWS_PER_SUB, HID), jnp.float32)
    aggp = _sc_segment_sum(p1, src1d, dst1d, zeros_hid, nbuf=2,
                           chunk=CHUNK, idxb=IDXB, tc_tiling=False)

    p2, r2 = _stage2(aggp, degp, r1, W2_l, W2_r, b2_l.reshape(1, OUT))

    zeros_out = jnp.zeros((ROWS_PER_SUB, OUT), jnp.float32)
    agg2p = _sc_segment_sum(p2, src1d, dst1d, zeros_out, nbuf=4,
                            chunk=CHUNK, idxb=IDXB, tc_tiling=False)

    return _stage3(agg2p, degp, r2)[:N]


# speedup vs baseline: 1.1564x; 1.0931x over previous
import functools

import jax
import jax.numpy as jnp
from jax import lax
from jax.experimental import pallas as pl
from jax.experimental.pallas import tpu as pltpu
from jax.experimental.pallas import tpu_sc as plsc

N = 10000
NP = 10112
DEG_W = 8
E = 320000
FEAT = 128
EMB = 64
HID = 128
OUT = 64

NC = 2
NS = 16
NW = NC * NS
EDGES_PER_W = E // NW
CHUNK = 125
CHUNKS_PER_W = EDGES_PER_W // CHUNK
IDXB = 16
ROWS_PER_SUB = NP // NS

_MESH = plsc.VectorSubcoreMesh(
    core_axis_name="c", subcore_axis_name="s", num_cores=NC, num_subcores=NS
)


def _sc_degree(dst2d, zeros_deg, ones_tile):
    @functools.partial(
        pl.kernel,
        out_type=jax.ShapeDtypeStruct((NC, NP, DEG_W), jnp.float32),
        mesh=_MESH,
        scratch_types=[
            pltpu.VMEM((IDXB, CHUNK), jnp.int32),
            pltpu.VMEM((CHUNK, DEG_W), jnp.float32),
            pltpu.VMEM_SHARED((NP, DEG_W), jnp.float32),
            pltpu.SemaphoreType.DMA,
            pltpu.SemaphoreType.DMA,
        ],
        compiler_params=pltpu.CompilerParams(use_tc_tiling_on_sc=False),
    )
    def deg_kernel(dst_hbm, zdeg_hbm, ones_hbm, deg_hbm,
                   dst_v, ones_v, sh_deg, sem_a, sem_b):
        cid = lax.axis_index("c")
        sid = lax.axis_index("s")
        g = cid * NS + sid

        pltpu.sync_copy(
            zdeg_hbm, sh_deg.at[pl.ds(sid * ROWS_PER_SUB, ROWS_PER_SUB)]
        )
        pltpu.sync_copy(ones_hbm, ones_v)
        plsc.subcore_barrier()

        @pl.loop(0, CHUNKS_PER_W // IDXB)
        def _(blk):
            pltpu.sync_copy(dst_hbm.at[g, pl.ds(blk * IDXB, IDXB)], dst_v)

            @pl.loop(0, IDXB, step=2)
            def _(c):
                d1 = pltpu.async_copy(
                    ones_v, sh_deg.at[dst_v.at[c]], sem_a, add=True
                )
                d2 = pltpu.async_copy(
                    ones_v, sh_deg.at[dst_v.at[c + 1]], sem_b, add=True
                )
                d1.wait()
                d2.wait()

        plsc.subcore_barrier()
        sl = pl.ds(sid * ROWS_PER_SUB, ROWS_PER_SUB)
        pltpu.sync_copy(sh_deg.at[sl], deg_hbm.at[cid, sl])

    return deg_kernel(dst2d, zeros_deg, ones_tile)


def _sc_segment_sum(p, src1d, dst1d, zeros_d, nbuf, chunk, idxb, tc_tiling):
    D = p.shape[1]
    chunks_per_w = EDGES_PER_W // chunk
    src2d = src1d.reshape(NW, chunks_per_w, chunk)
    dst2d = dst1d.reshape(NW, chunks_per_w, chunk)

    scratch = [
        pltpu.VMEM((idxb, chunk), jnp.int32),
        pltpu.VMEM((idxb, chunk), jnp.int32),
    ]
    scratch += [pltpu.VMEM((chunk, D), jnp.float32) for _ in range(nbuf)]
    scratch += [pltpu.VMEM_SHARED((NP, D), jnp.float32)]
    scratch += [pltpu.SemaphoreType.DMA for _ in range(nbuf)]

    @functools.partial(
        pl.kernel,
        out_type=jax.ShapeDtypeStruct((NC, NP, D), jnp.float32),
        mesh=_MESH,
        scratch_types=scratch,
        compiler_params=None if tc_tiling else pltpu.CompilerParams(
            use_tc_tiling_on_sc=False),
    )
    def seg_kernel(p_hbm, src_hbm, dst_hbm, z_hbm, agg_hbm,
                   src_v, dst_v, *rows_and_sems):
        rows = rows_and_sems[:nbuf]
        sh_agg = rows_and_sems[nbuf]
        sems = rows_and_sems[nbuf + 1:]
        cid = lax.axis_index("c")
        sid = lax.axis_index("s")
        g = cid * NS + sid

        pltpu.sync_copy(z_hbm, sh_agg.at[pl.ds(sid * ROWS_PER_SUB, ROWS_PER_SUB)])
        plsc.subcore_barrier()

        def prime(blk):
            pltpu.sync_copy(src_hbm.at[g, pl.ds(blk * idxb, idxb)], src_v)
            pltpu.sync_copy(dst_hbm.at[g, pl.ds(blk * idxb, idxb)], dst_v)
            for b in range(nbuf):
                pltpu.async_copy(p_hbm.at[src_v.at[b]], rows[b], sems[b])

        prime(0)

        @pl.loop(0, chunks_per_w // idxb)
        def _(blk):
            @pl.loop(0, idxb, step=nbuf)
            def _(c):
                for b in range(nbuf):
                    j = c + b
                    pltpu.make_async_copy(
                        p_hbm.at[src_v.at[j]], rows[b], sems[b]
                    ).wait()
                    pltpu.sync_copy(rows[b], sh_agg.at[dst_v.at[j]], add=True)

                    @pl.when(j < idxb - nbuf)
                    def _():
                        pltpu.async_copy(
                            p_hbm.at[src_v.at[j + nbuf]], rows[b], sems[b]
                        )

            @pl.when(blk < chunks_per_w // idxb - 1)
            def _():
                prime(blk + 1)

        plsc.subcore_barrier()

        sl = pl.ds(sid * ROWS_PER_SUB, ROWS_PER_SUB)
        pltpu.sync_copy(sh_agg.at[sl], agg_hbm.at[cid, sl])

    return seg_kernel(p, src2d, dst2d, zeros_d)


ROW_BLK = 2528
GRID = NP // ROW_BLK


def _matT(a, w):
    return lax.dot_general(a, w, (((1,), (1,)), ((), ())),
                           preferred_element_type=jnp.float32)


def _stage1(x, emb, W1_l, W1_r, b1):
    def body(x_ref, e_ref, wl_ref, wr_ref, b_ref, p1_ref, r1_ref):
        xv = x_ref[...]
        ev = e_ref[...]
        wl = wl_ref[...]
        wr = wr_ref[...]
        p1_ref[...] = _matT(xv, wl[:, :FEAT]) + _matT(ev, wl[:, FEAT:])
        r1_ref[...] = (_matT(xv, wr[:, :FEAT]) + _matT(ev, wr[:, FEAT:])
                       + b_ref[...])

    full = lambda s: pl.BlockSpec(s, lambda i: (0,) * len(s))
    return pl.pallas_call(
        body,
        grid=(GRID,),
        in_specs=[
            pl.BlockSpec((ROW_BLK, FEAT), lambda i: (i, 0)),
            pl.BlockSpec((ROW_BLK, EMB), lambda i: (i, 0)),
            full((HID, FEAT + EMB)), full((HID, FEAT + EMB)),
            full((1, HID)),
        ],
        out_specs=[
            pl.BlockSpec((ROW_BLK, HID), lambda i: (i, 0)),
            pl.BlockSpec((ROW_BLK, HID), lambda i: (i, 0)),
        ],
        out_shape=[
            jax.ShapeDtypeStruct((NP, HID), jnp.float32),
            jax.ShapeDtypeStruct((NP, HID), jnp.float32),
        ],
    )(x, emb, W1_l, W1_r, b1)


def _stage2(aggp, degp, r1, w2l, w2r, b2):
    def body(agg_ref, deg_ref, r1_ref, w2l_ref, w2r_ref, b_ref,
             p2_ref, r2_ref):
        aggs = agg_ref[...]
        degs = deg_ref[...]
        deg = degs[0, :, 0:1] + degs[1, :, 0:1]
        inv = 1.0 / jnp.maximum(deg, 1.0)
        h1 = jnp.maximum((aggs[0] + aggs[1]) * inv + r1_ref[...], 0.0)
        p2_ref[...] = _matT(h1, w2l_ref[...])
        r2_ref[...] = _matT(h1, w2r_ref[...]) + b_ref[...]

    full = lambda s: pl.BlockSpec(s, lambda i: (0,) * len(s))
    return pl.pallas_call(
        body,
        grid=(GRID,),
        in_specs=[
            pl.BlockSpec((NC, ROW_BLK, HID), lambda i: (0, i, 0)),
            pl.BlockSpec((NC, ROW_BLK, DEG_W), lambda i: (0, i, 0)),
            pl.BlockSpec((ROW_BLK, HID), lambda i: (i, 0)),
            full((OUT, HID)), full((OUT, HID)), full((1, OUT)),
        ],
        out_specs=[
            pl.BlockSpec((ROW_BLK, OUT), lambda i: (i, 0)),
            pl.BlockSpec((ROW_BLK, OUT), lambda i: (i, 0)),
        ],
        out_shape=[
            jax.ShapeDtypeStruct((NP, OUT), jnp.float32),
            jax.ShapeDtypeStruct((NP, OUT), jnp.float32),
        ],
    )(aggp, degp, r1, w2l, w2r, b2)


def _stage3(agg2p, degp, r2):
    def body(agg_ref, deg_ref, r2_ref, out_ref):
        aggs = agg_ref[...]
        degs = deg_ref[...]
        deg = degs[0, :, 0:1] + degs[1, :, 0:1]
        inv = 1.0 / jnp.maximum(deg, 1.0)
        out_ref[...] = (aggs[0] + aggs[1]) * inv + r2_ref[...]

    return pl.pallas_call(
        body,
        grid=(GRID,),
        in_specs=[
            pl.BlockSpec((NC, ROW_BLK, OUT), lambda i: (0, i, 0)),
            pl.BlockSpec((NC, ROW_BLK, DEG_W), lambda i: (0, i, 0)),
            pl.BlockSpec((ROW_BLK, OUT), lambda i: (i, 0)),
        ],
        out_specs=pl.BlockSpec((ROW_BLK, OUT), lambda i: (i, 0)),
        out_shape=jax.ShapeDtypeStruct((N, OUT), jnp.float32),
    )(agg2p, degp, r2)


@jax.jit
def kernel(x, edge_index, emb, W1_l, b1_l, W1_r, W2_l, b2_l, W2_r):
    src1d = edge_index[0].astype(jnp.int32)
    dst1d = edge_index[1].astype(jnp.int32)

    zeros_deg = jnp.zeros((ROWS_PER_SUB, DEG_W), jnp.float32)
    ones_tile = jnp.ones((CHUNK, DEG_W), jnp.float32)
    degp = _sc_degree(dst1d.reshape(NW, CHUNKS_PER_W, CHUNK), zeros_deg, ones_tile)

    p1, r1 = _stage1(x, emb, W1_l, W1_r, b1_l.reshape(1, HID))

    zeros_hid = jnp.zeros((ROWS_PER_SUB, HID), jnp.float32)
    aggp = _sc_segment_sum(p1, src1d, dst1d, zeros_hid, nbuf=2,
                           chunk=CHUNK, idxb=IDXB, tc_tiling=False)

    p2, r2 = _stage2(aggp, degp, r1, W2_l, W2_r, b2_l.reshape(1, OUT))

    zeros_out = jnp.zeros((ROWS_PER_SUB, OUT), jnp.float32)
    agg2p = _sc_segment_sum(p2, src1d, dst1d, zeros_out, nbuf=4,
                            chunk=CHUNK, idxb=IDXB, tc_tiling=False)

    return _stage3(agg2p, degp, r2)
